# baseline (speedup 1.0000x reference)
import functools

import jax
import jax.numpy as jnp
from jax import lax
from jax.experimental import pallas as pl
from jax.experimental.pallas import tpu as pltpu
from jax.experimental.pallas import tpu_sc as plsc

D_MODEL = 64
SEQ = 200
BATCH = 4096
ROWS = SEQ * BATCH
NC = 2
NS = 16
NW = NC * NS
PER_W = ROWS // NW
KC = 128
NB = 4
NHALF = 2
NCH_H = PER_W // KC // NHALF
NPOS = 600
LANES = 16


def _emb_body(seq_hbm, p1_hbm, p2_hbm, wtab_hbm, ptab_hbm, out_hbm,
              idx_w, idx_p1, idx_p2, idx_out, ptab_v, b0, b1, b2, b3, sems):
    wid = lax.axis_index("s") * NC + lax.axis_index("c")
    bufs = [b0, b1, b2, b3]
    iota = lax.iota(jnp.int32, LANES)

    pltpu.sync_copy(ptab_hbm, ptab_v)

    def drain(b):
        pltpu.make_async_copy(out_hbm.at[pl.ds(0, KC)], bufs[b],
                              sems.at[b]).wait()

    for h in range(NHALF):
        idxrow = (wid * NHALF + h) * NCH_H
        base = (wid * NHALF + h) * NCH_H * KC
        pltpu.sync_copy(seq_hbm.at[pl.ds(idxrow, NCH_H)], idx_w)
        pltpu.sync_copy(p1_hbm.at[pl.ds(idxrow, NCH_H)], idx_p1)
        pltpu.sync_copy(p2_hbm.at[pl.ds(idxrow, NCH_H)], idx_p2)

        def addpos(c, b):
            def grp(g, _):
                row0 = g * LANES
                p1v = idx_p1[c, pl.ds(row0, LANES)]
                p2v = idx_p2[c, pl.ds(row0, LANES)]
                for j in range(LANES):
                    p1 = p1v[j]
                    p2 = p2v[j]
                    for k in range(D_MODEL // LANES):
                        sl = pl.ds(k * LANES, LANES)
                        bufs[b][row0 + j, sl] = (bufs[b][row0 + j, sl]
                                                 + ptab_v[p1, sl]
                                                 + ptab_v[p2, sl])
                return 0

            lax.fori_loop(0, KC // LANES, grp, 0, unroll=2)

        def step(t, b):
            @pl.when(t < NCH_H)
            def _():
                @pl.when(t >= NB)
                def _():
                    drain(b)

                pltpu.async_copy(wtab_hbm.at[idx_w.at[t]], bufs[b],
                                 sems.at[b])

            @pl.when((t >= 1) & (t < NCH_H + 1))
            def _():
                b1_ = (b - 1) % NB
                drain(b1_)
                addpos(t - 1, b1_)
                tok0 = base + (t - 1) * KC
                for v in range(KC // LANES):
                    tok = tok0 + v * LANES + iota
                    orow = ((tok & (BATCH - 1)) * SEQ) + (tok >> 12)
                    idx_out[b1_, pl.ds(v * LANES, LANES)] = orow * 2
                pltpu.async_copy(bufs[b1_], out_hbm.at[idx_out.at[b1_]],
                                 sems.at[b1_])

        def outer(g, _):
            for b in range(NB):
                step(g * NB + b, b)
            return 0

        lax.fori_loop(0, (NCH_H + 1 + NB - 1) // NB, outer, 0)

        for b in range(NB):
            drain(b)


@jax.jit
def _run(seq2d, p1_2d, p2_2d, wtab, ptab):
    mesh = plsc.VectorSubcoreMesh(
        core_axis_name="c", subcore_axis_name="s",
        num_cores=NC, num_subcores=NS)
    return pl.kernel(
        _emb_body,
        out_type=jax.ShapeDtypeStruct((2 * ROWS, D_MODEL), jnp.float32),
        mesh=mesh,
        scratch_types=[
            pltpu.VMEM((NCH_H, KC), jnp.int32),
            pltpu.VMEM((NCH_H, KC), jnp.int32),
            pltpu.VMEM((NCH_H, KC), jnp.int32),
            pltpu.VMEM((NB, KC), jnp.int32),
            pltpu.VMEM((NPOS, D_MODEL), jnp.float32),
            pltpu.VMEM((KC, D_MODEL), jnp.float32),
            pltpu.VMEM((KC, D_MODEL), jnp.float32),
            pltpu.VMEM((KC, D_MODEL), jnp.float32),
            pltpu.VMEM((KC, D_MODEL), jnp.float32),
            pltpu.SemaphoreType.DMA((NB,)),
        ],
        compiler_params=pltpu.CompilerParams(use_tc_tiling_on_sc=False),
    )(seq2d, p1_2d, p2_2d, wtab, ptab)


def kernel(sentences_seq, position_to_entity1_batch, position_to_entity2_batch,
           word_embedding, position_embedding):
    seq2d = sentences_seq.reshape(ROWS // KC, KC).astype(jnp.int32)
    p1_2d = position_to_entity1_batch.reshape(ROWS // KC, KC).astype(jnp.int32)
    p2_2d = position_to_entity2_batch.reshape(ROWS // KC, KC).astype(jnp.int32)
    out = _run(seq2d, p1_2d, p2_2d,
               word_embedding.astype(jnp.float32),
               position_embedding.astype(jnp.float32))
    return out.reshape(BATCH, SEQ, 2 * D_MODEL)[:, :, :D_MODEL]


# speedup vs baseline: 1.0527x; 1.0527x over previous
import functools

import jax
import jax.numpy as jnp
from jax import lax
from jax.experimental import pallas as pl
from jax.experimental.pallas import tpu as pltpu
from jax.experimental.pallas import tpu_sc as plsc

D_MODEL = 64
SEQ = 200
BATCH = 4096
ROWS = SEQ * BATCH
NC = 2
NS = 16
NW = NC * NS
PER_W = ROWS // NW
KC = 256
SPLIT = 2
IDXW = KC // SPLIT
NB = 4
NHALF = 2
NCH_H = PER_W // KC // NHALF
LANES = 16


def _emb_body(seq_hbm, p1_hbm, p2_hbm, wtab_hbm, ptab_hbm, out_hbm,
              idx_w, idx_p1, idx_p2, idx_out, b0, b1, b2, b3, sems):
    wid = lax.axis_index("s") * NC + lax.axis_index("c")
    bufs = [b0, b1, b2, b3]
    iota = lax.iota(jnp.int32, LANES)

    def drain(b):
        pltpu.make_async_copy(out_hbm.at[pl.ds(0, KC)], bufs[b],
                              sems.at[b]).wait()

    for h in range(NHALF):
        idxrow = (wid * NHALF + h) * NCH_H * SPLIT
        base = (wid * NHALF + h) * NCH_H * KC
        pltpu.sync_copy(seq_hbm.at[pl.ds(idxrow, NCH_H * SPLIT)], idx_w)
        pltpu.sync_copy(p1_hbm.at[pl.ds(idxrow, NCH_H * SPLIT)], idx_p1)
        pltpu.sync_copy(p2_hbm.at[pl.ds(idxrow, NCH_H * SPLIT)], idx_p2)

        def step(t, b):
            @pl.when((t >= 3) & (t < NCH_H + 3))
            def _():
                b3 = (b - 3) % NB
                drain(b3)
                tok0 = base + (t - 3) * KC
                for q in range(SPLIT):
                    for v in range(IDXW // LANES):
                        tok = tok0 + q * IDXW + v * LANES + iota
                        orow = ((tok & (BATCH - 1)) * SEQ) + (tok >> 12)
                        idx_out[b3 * SPLIT + q,
                                pl.ds(v * LANES, LANES)] = orow * 2
                for q in range(SPLIT):
                    pltpu.async_copy(
                        bufs[b3].at[pl.ds(q * IDXW, IDXW)],
                        out_hbm.at[idx_out.at[b3 * SPLIT + q]],
                        sems.at[b3])

            @pl.when((t >= 2) & (t < NCH_H + 2))
            def _():
                b2_ = (b - 2) % NB
                drain(b2_)
                for q in range(SPLIT):
                    pltpu.async_copy(
                        ptab_hbm.at[idx_p2.at[(t - 2) * SPLIT + q]],
                        bufs[b2_].at[pl.ds(q * IDXW, IDXW)],
                        sems.at[b2_], add=True)

            @pl.when((t >= 1) & (t < NCH_H + 1))
            def _():
                b1_ = (b - 1) % NB
                drain(b1_)
                for q in range(SPLIT):
                    pltpu.async_copy(
                        ptab_hbm.at[idx_p1.at[(t - 1) * SPLIT + q]],
                        bufs[b1_].at[pl.ds(q * IDXW, IDXW)],
                        sems.at[b1_], add=True)

            @pl.when(t < NCH_H)
            def _():
                @pl.when(t >= NB)
                def _():
                    drain(b)

                for q in range(SPLIT):
                    pltpu.async_copy(
                        wtab_hbm.at[idx_w.at[t * SPLIT + q]],
                        bufs[b].at[pl.ds(q * IDXW, IDXW)],
                        sems.at[b])

        def outer(g, _):
            for b in range(NB):
                step(g * NB + b, b)
            return 0

        lax.fori_loop(0, (NCH_H + 3 + NB - 1) // NB, outer, 0)

        for b in range(NB):
            drain(b)


@jax.jit
def _run(seq2d, p1_2d, p2_2d, wtab, ptab):
    mesh = plsc.VectorSubcoreMesh(
        core_axis_name="c", subcore_axis_name="s",
        num_cores=NC, num_subcores=NS)
    return pl.kernel(
        _emb_body,
        out_type=jax.ShapeDtypeStruct((2 * ROWS, D_MODEL), jnp.float32),
        mesh=mesh,
        scratch_types=[
            pltpu.VMEM((NCH_H * SPLIT, IDXW), jnp.int32),
            pltpu.VMEM((NCH_H * SPLIT, IDXW), jnp.int32),
            pltpu.VMEM((NCH_H * SPLIT, IDXW), jnp.int32),
            pltpu.VMEM((NB * SPLIT, IDXW), jnp.int32),
            pltpu.VMEM((KC, D_MODEL), jnp.float32),
            pltpu.VMEM((KC, D_MODEL), jnp.float32),
            pltpu.VMEM((KC, D_MODEL), jnp.float32),
            pltpu.VMEM((KC, D_MODEL), jnp.float32),
            pltpu.SemaphoreType.DMA((NB,)),
        ],
        compiler_params=pltpu.CompilerParams(use_tc_tiling_on_sc=False),
    )(seq2d, p1_2d, p2_2d, wtab, ptab)


def kernel(sentences_seq, position_to_entity1_batch, position_to_entity2_batch,
           word_embedding, position_embedding):
    seq2d = sentences_seq.reshape(ROWS // IDXW, IDXW).astype(jnp.int32)
    p1_2d = position_to_entity1_batch.reshape(ROWS // IDXW, IDXW).astype(jnp.int32)
    p2_2d = position_to_entity2_batch.reshape(ROWS // IDXW, IDXW).astype(jnp.int32)
    out = _run(seq2d, p1_2d, p2_2d,
               word_embedding.astype(jnp.float32),
               position_embedding.astype(jnp.float32))
    return out.reshape(BATCH, SEQ, 2 * D_MODEL)[:, :, :D_MODEL]
